# Initial kernel scaffold; baseline (speedup 1.0000x reference)
import jax
import jax.numpy as jnp
from jax.experimental import pallas as pl


def kernel(categorical_features, numerical_features, emb_tables, W1, b1, W2, b2):
    raise NotImplementedError("write your pallas kernel here")


# speedup vs baseline: 2.0590x; 2.0590x over previous
import functools

import jax
import jax.numpy as jnp
from jax import lax
from jax.experimental import pallas as pl
from jax.experimental.pallas import tpu as pltpu
from jax.experimental.pallas import tpu_sc as plsc

NUM_FIELDS = 26
VOCAB = 100000
EMB_DIM = 32
NUM_NUMERICAL = 13
OUTPUT_DIM = 64
BATCH = 16384
TOTAL_ROWS = BATCH * NUM_FIELDS
CAT_DIM = NUM_FIELDS * EMB_DIM

SLICE = 128
GROUP = 8


@functools.lru_cache(maxsize=None)
def _make_gather(num_cores: int, num_subcores: int):
    nw = num_cores * num_subcores
    b_per_w = TOTAL_ROWS // nw
    n_slices = b_per_w // SLICE
    n_groups = n_slices // GROUP
    group_rows = GROUP * SLICE

    mesh = plsc.VectorSubcoreMesh(core_axis_name="c", subcore_axis_name="s")

    @functools.partial(
        pl.kernel,
        mesh=mesh,
        compiler_params=pltpu.CompilerParams(use_tc_tiling_on_sc=False),
        out_type=jax.ShapeDtypeStruct((TOTAL_ROWS, EMB_DIM), jnp.float32),
        scratch_types=[
            pltpu.VMEM((n_slices, SLICE), jnp.int32),
            pltpu.VMEM((group_rows, EMB_DIM), jnp.float32),
            pltpu.SemaphoreType.DMA,
        ],
    )
    def gather_kernel(table_hbm, idx_hbm, out_hbm, idx_v, rows_v, sem):
        wid = lax.axis_index("s") * num_cores + lax.axis_index("c")
        pltpu.sync_copy(idx_hbm.at[pl.ds(wid * n_slices, n_slices)], idx_v)

        def body(g, carry):
            handles = []
            for j in range(GROUP):
                handles.append(pltpu.async_copy(
                    table_hbm.at[idx_v.at[g * GROUP + j]],
                    rows_v.at[pl.ds(j * SLICE, SLICE)],
                    sem,
                ))
            for h in handles:
                h.wait()
            pltpu.sync_copy(
                rows_v,
                out_hbm.at[pl.ds(wid * b_per_w + g * group_rows, group_rows)],
            )
            return carry

        lax.fori_loop(0, n_groups, body, 0)

    return gather_kernel


def _mlp_body(g_ref, n_ref, w1_ref, b1_ref, w2a_ref, w2n_ref, b2_ref, o_ref):
    h = jnp.maximum(
        jnp.dot(n_ref[...], w1_ref[...], preferred_element_type=jnp.float32)
        + b1_ref[...], 0.0)
    acc = (jnp.dot(g_ref[...], w2a_ref[...], preferred_element_type=jnp.float32)
           + jnp.dot(h, w2n_ref[...], preferred_element_type=jnp.float32)
           + b2_ref[...])
    o_ref[...] = jnp.maximum(acc, 0.0)


def _mlp(gathered, numerical, w1, b1, w2a, w2n, b2):
    bm = 2048
    grid = (BATCH // bm,)
    nn = numerical.shape[1]
    return pl.pallas_call(
        _mlp_body,
        grid=grid,
        in_specs=[
            pl.BlockSpec((bm, CAT_DIM), lambda i: (i, 0)),
            pl.BlockSpec((bm, nn), lambda i: (i, 0)),
            pl.BlockSpec((nn, EMB_DIM), lambda i: (0, 0)),
            pl.BlockSpec((1, EMB_DIM), lambda i: (0, 0)),
            pl.BlockSpec((CAT_DIM, OUTPUT_DIM), lambda i: (0, 0)),
            pl.BlockSpec((EMB_DIM, OUTPUT_DIM), lambda i: (0, 0)),
            pl.BlockSpec((1, OUTPUT_DIM), lambda i: (0, 0)),
        ],
        out_specs=pl.BlockSpec((bm, OUTPUT_DIM), lambda i: (i, 0)),
        out_shape=jax.ShapeDtypeStruct((BATCH, OUTPUT_DIM), jnp.float32),
    )(gathered, numerical, w1, b1, w2a, w2n, b2)


def kernel(categorical_features, numerical_features, emb_tables, W1, b1, W2, b2):
    flat_table = emb_tables.reshape(NUM_FIELDS * VOCAB, EMB_DIM)
    offs = (jnp.arange(NUM_FIELDS, dtype=jnp.int32) * VOCAB)[None, :]
    flat_idx = (categorical_features.astype(jnp.int32) + offs).reshape(
        TOTAL_ROWS // SLICE, SLICE)

    info = plsc.get_sparse_core_info()
    gathered = _make_gather(info.num_cores, info.num_subcores)(
        flat_table, flat_idx)
    gathered = gathered.reshape(BATCH, CAT_DIM)

    num_pad = jnp.pad(numerical_features, ((0, 0), (0, 3)))
    w1_pad = jnp.pad(W1, ((0, 3), (0, 0)))

    return _mlp(gathered, num_pad, w1_pad, b1.reshape(1, EMB_DIM),
                W2[:CAT_DIM], W2[CAT_DIM:], b2.reshape(1, OUTPUT_DIM))


# speedup vs baseline: 6.4721x; 3.1433x over previous
import functools

import jax
import jax.numpy as jnp
from jax import lax
from jax.experimental import pallas as pl
from jax.experimental.pallas import tpu as pltpu
from jax.experimental.pallas import tpu_sc as plsc

NUM_FIELDS = 26
VOCAB = 100000
EMB_DIM = 32
NUM_NUMERICAL = 13
OUTPUT_DIM = 64
BATCH = 16384
CAT_DIM = NUM_FIELDS * EMB_DIM

HALF_B = BATCH // 2


@functools.lru_cache(maxsize=None)
def _make_lookup(num_cores: int, num_subcores: int):
    mesh = plsc.VectorSubcoreMesh(core_axis_name="c", subcore_axis_name="s")

    @functools.partial(
        pl.kernel,
        mesh=mesh,
        compiler_params=pltpu.CompilerParams(needs_layout_passes=False),
        out_type=jax.ShapeDtypeStruct((CAT_DIM, BATCH), jnp.float32),
        scratch_types=[
            pltpu.VMEM((VOCAB,), jnp.float32),
            pltpu.VMEM((HALF_B,), jnp.int32),
            pltpu.VMEM((HALF_B,), jnp.float32),
        ],
    )
    def lookup_kernel(table_hbm, idx_hbm, out_hbm, row_v, idx_v, out_v):
        d = lax.axis_index("s") * num_cores + lax.axis_index("c")

        def field_body(f, carry):
            pltpu.sync_copy(table_hbm.at[f, d], row_v)
            r = f * EMB_DIM + d
            for h in range(2):
                pltpu.sync_copy(idx_hbm.at[f, pl.ds(h * HALF_B, HALF_B)],
                                idx_v)

                def gather_body(i, c):
                    vec = idx_v[pl.ds(i * 16, 16)]
                    out_v[pl.ds(i * 16, 16)] = plsc.load_gather(row_v, [vec])
                    return c

                lax.fori_loop(0, HALF_B // 16, gather_body, 0)
                pltpu.sync_copy(out_v,
                                out_hbm.at[r, pl.ds(h * HALF_B, HALF_B)])
            return carry

        lax.fori_loop(0, NUM_FIELDS, field_body, 0)

    return lookup_kernel


def _mlp_body(g_ref, n_ref, w1_ref, b1_ref, w2a_ref, w2n_ref, b2_ref, o_ref):
    h = jnp.maximum(
        jnp.dot(n_ref[...], w1_ref[...], preferred_element_type=jnp.float32)
        + b1_ref[...], 0.0)
    acc = lax.dot_general(
        g_ref[...], w2a_ref[...], (((0,), (0,)), ((), ())),
        preferred_element_type=jnp.float32)
    acc = acc + jnp.dot(h, w2n_ref[...], preferred_element_type=jnp.float32)
    acc = acc + b2_ref[...]
    o_ref[...] = jnp.maximum(acc, 0.0)


def _mlp(gt, numerical, w1, b1, w2a, w2n, b2):
    bm = 2048
    grid = (BATCH // bm,)
    nn = numerical.shape[1]
    return pl.pallas_call(
        _mlp_body,
        grid=grid,
        in_specs=[
            pl.BlockSpec((CAT_DIM, bm), lambda i: (0, i)),
            pl.BlockSpec((bm, nn), lambda i: (i, 0)),
            pl.BlockSpec((nn, EMB_DIM), lambda i: (0, 0)),
            pl.BlockSpec((1, EMB_DIM), lambda i: (0, 0)),
            pl.BlockSpec((CAT_DIM, OUTPUT_DIM), lambda i: (0, 0)),
            pl.BlockSpec((EMB_DIM, OUTPUT_DIM), lambda i: (0, 0)),
            pl.BlockSpec((1, OUTPUT_DIM), lambda i: (0, 0)),
        ],
        out_specs=pl.BlockSpec((bm, OUTPUT_DIM), lambda i: (i, 0)),
        out_shape=jax.ShapeDtypeStruct((BATCH, OUTPUT_DIM), jnp.float32),
    )(gt, numerical, w1, b1, w2a, w2n, b2)


def kernel(categorical_features, numerical_features, emb_tables, W1, b1, W2, b2):
    table_t = emb_tables.transpose(0, 2, 1)
    idx_t = categorical_features.astype(jnp.int32).T

    info = plsc.get_sparse_core_info()
    gt = _make_lookup(info.num_cores, info.num_subcores)(table_t, idx_t)

    num_pad = jnp.pad(numerical_features, ((0, 0), (0, 3)))
    w1_pad = jnp.pad(W1, ((0, 3), (0, 0)))

    return _mlp(gt, num_pad, w1_pad, b1.reshape(1, EMB_DIM),
                W2[:CAT_DIM], W2[CAT_DIM:], b2.reshape(1, OUTPUT_DIM))


# speedup vs baseline: 9.0286x; 1.3950x over previous
import functools

import jax
import jax.numpy as jnp
from jax import lax
from jax.experimental import pallas as pl
from jax.experimental.pallas import tpu as pltpu
from jax.experimental.pallas import tpu_sc as plsc

NUM_FIELDS = 26
VOCAB = 100000
EMB_DIM = 32
NUM_NUMERICAL = 13
OUTPUT_DIM = 64
BATCH = 16384
CAT_DIM = NUM_FIELDS * EMB_DIM

HALF_B = BATCH // 2


@functools.lru_cache(maxsize=None)
def _make_lookup(num_cores: int, num_subcores: int):
    mesh = plsc.VectorSubcoreMesh(core_axis_name="c", subcore_axis_name="s")

    @functools.partial(
        pl.kernel,
        mesh=mesh,
        compiler_params=pltpu.CompilerParams(needs_layout_passes=False),
        out_type=jax.ShapeDtypeStruct((CAT_DIM, BATCH), jnp.float32),
        scratch_types=[
            pltpu.VMEM((VOCAB,), jnp.float32),
            pltpu.VMEM((HALF_B,), jnp.int32),
            pltpu.VMEM((HALF_B,), jnp.float32),
        ],
    )
    def lookup_kernel(table_hbm, idx_hbm, out_hbm, row_v, idx_v, out_v):
        d = lax.axis_index("s") * num_cores + lax.axis_index("c")

        def field_body(f, carry):
            pltpu.sync_copy(table_hbm.at[f, d], row_v)
            r = f * EMB_DIM + d
            for h in range(2):
                pltpu.sync_copy(idx_hbm.at[f, pl.ds(h * HALF_B, HALF_B)],
                                idx_v)

                @plsc.parallel_loop(0, HALF_B, 16, unroll=8)
                def gather_body(i):
                    vec = idx_v[pl.ds(i, 16)]
                    out_v[pl.ds(i, 16)] = plsc.load_gather(row_v, [vec])
                pltpu.sync_copy(out_v,
                                out_hbm.at[r, pl.ds(h * HALF_B, HALF_B)])
            return carry

        lax.fori_loop(0, NUM_FIELDS, field_body, 0)

    return lookup_kernel


def _mlp_body(g_ref, n_ref, w1_ref, b1_ref, w2a_ref, w2n_ref, b2_ref, o_ref):
    h = jnp.maximum(
        jnp.dot(n_ref[...], w1_ref[...], preferred_element_type=jnp.float32)
        + b1_ref[...], 0.0)
    acc = lax.dot_general(
        g_ref[...], w2a_ref[...], (((0,), (0,)), ((), ())),
        preferred_element_type=jnp.float32)
    acc = acc + jnp.dot(h, w2n_ref[...], preferred_element_type=jnp.float32)
    acc = acc + b2_ref[...]
    o_ref[...] = jnp.maximum(acc, 0.0)


def _mlp(gt, numerical, w1, b1, w2a, w2n, b2):
    bm = 2048
    grid = (BATCH // bm,)
    nn = numerical.shape[1]
    return pl.pallas_call(
        _mlp_body,
        grid=grid,
        in_specs=[
            pl.BlockSpec((CAT_DIM, bm), lambda i: (0, i)),
            pl.BlockSpec((bm, nn), lambda i: (i, 0)),
            pl.BlockSpec((nn, EMB_DIM), lambda i: (0, 0)),
            pl.BlockSpec((1, EMB_DIM), lambda i: (0, 0)),
            pl.BlockSpec((CAT_DIM, OUTPUT_DIM), lambda i: (0, 0)),
            pl.BlockSpec((EMB_DIM, OUTPUT_DIM), lambda i: (0, 0)),
            pl.BlockSpec((1, OUTPUT_DIM), lambda i: (0, 0)),
        ],
        out_specs=pl.BlockSpec((bm, OUTPUT_DIM), lambda i: (i, 0)),
        out_shape=jax.ShapeDtypeStruct((BATCH, OUTPUT_DIM), jnp.float32),
    )(gt, numerical, w1, b1, w2a, w2n, b2)


def kernel(categorical_features, numerical_features, emb_tables, W1, b1, W2, b2):
    table_t = emb_tables.transpose(0, 2, 1)
    idx_t = categorical_features.astype(jnp.int32).T

    info = plsc.get_sparse_core_info()
    gt = _make_lookup(info.num_cores, info.num_subcores)(table_t, idx_t)

    num_pad = jnp.pad(numerical_features, ((0, 0), (0, 3)))
    w1_pad = jnp.pad(W1, ((0, 3), (0, 0)))

    return _mlp(gt, num_pad, w1_pad, b1.reshape(1, EMB_DIM),
                W2[:CAT_DIM], W2[CAT_DIM:], b2.reshape(1, OUTPUT_DIM))
